# baseline (speedup 1.0000x reference)
import functools

import jax
import jax.numpy as jnp
from jax import lax
from jax.experimental import pallas as pl
from jax.experimental.pallas import tpu as pltpu
from jax.experimental.pallas import tpu_sc as plsc

VOCAB = 1000000
D = 32
B = 4096
S = 200
T = 1024
NC, NS = 2, 16
NW = NC * NS
BW = B // NW
HALF = S // 2
NHR = 2 * BW
NBUF = 4
G = VOCAB // 4
NBLK = VOCAB // 128
BLK_PER_W = NBLK // NW
BLK_REM = NBLK % NW


def _sc_regroup(table_t, tail64):
    mesh = plsc.VectorSubcoreMesh(
        core_axis_name="c", subcore_axis_name="s", num_cores=NC, num_subcores=NS
    )

    @functools.partial(
        pl.kernel,
        out_type=jax.ShapeDtypeStruct((G, 128), jnp.float32),
        mesh=mesh,
        scratch_types=[
            pltpu.VMEM((32, 128), jnp.float32),
            pltpu.VMEM((32, 128), jnp.float32),
            pltpu.VMEM((32, 128), jnp.float32),
            pltpu.VMEM((32, 128), jnp.float32),
            pltpu.VMEM((64, 32), jnp.float32),
            pltpu.SemaphoreType.DMA,
            pltpu.SemaphoreType.DMA,
            pltpu.SemaphoreType.DMA,
            pltpu.SemaphoreType.DMA,
        ],
        compiler_params=pltpu.CompilerParams(
            use_tc_tiling_on_sc=True, needs_layout_passes=False
        ),
    )
    def k(tt_hbm, tail_hbm, out_hbm, w0, w1, o0, o1, tl_v,
          gsem0, gsem1, osem0, osem1):
        wins = (w0, w1)
        obufs = (o0, o1)
        gsems = (gsem0, gsem1)
        osems = (osem0, osem1)
        iota = lax.iota(jnp.int32, 16)
        wid = lax.axis_index("s") * NC + lax.axis_index("c")
        blk0 = wid * BLK_PER_W

        def issue_in(blk, buf):
            pltpu.async_copy(
                tt_hbm.at[:, pl.ds(blk * 128, 128)], wins[buf], gsems[buf]
            )

        def wait_in(buf):
            pltpu.make_async_copy(
                tt_hbm.at[:, pl.ds(0, 128)], wins[buf], gsems[buf]
            ).wait()

        def transpose_block(buf):
            w = wins[buf]
            ob = obufs[buf]
            for ggl in range(32):
                for m in range(8):
                    rows = iota + (16 * (m % 2))
                    cols = jnp.full((16,), 4 * ggl + m // 2, jnp.int32)
                    ob[ggl, pl.ds(16 * m, 16)] = plsc.load_gather(w, [rows, cols])

        def issue_out(blk, buf):
            pltpu.async_copy(
                obufs[buf], out_hbm.at[pl.ds(blk * 32, 32)], osems[buf]
            )

        def wait_out(buf):
            pltpu.make_async_copy(
                obufs[buf], out_hbm.at[pl.ds(0, 32)], osems[buf]
            ).wait()

        issue_in(blk0, 0)
        issue_in(blk0 + 1, 1)
        for i in range(2):
            wait_in(i)
            transpose_block(i)
            issue_out(blk0 + i, i)
            issue_in(blk0 + i + 2, i)

        @pl.loop(0, BLK_PER_W - 4, step=2)
        def _(j0):
            i = j0 + 2
            for buf in range(2):
                wait_in(buf)
                wait_out(buf)
                transpose_block(buf)
                issue_out(blk0 + i + buf, buf)
                issue_in(blk0 + i + buf + 2, buf)

        for buf in range(2):
            wait_in(buf)
            wait_out(buf)
            transpose_block(buf)
            issue_out(blk0 + BLK_PER_W - 2 + buf, buf)
        wait_out(0)
        wait_out(1)

        @pl.when(wid < NBLK - NW * BLK_PER_W)
        def _():
            blk = NW * BLK_PER_W + wid
            issue_in(blk, 0)
            wait_in(0)
            transpose_block(0)
            issue_out(blk, 0)
            wait_out(0)

        @pl.when(wid == 0)
        def _():
            pltpu.sync_copy(tail_hbm, tl_v)
            ob = obufs[0]
            for ggl in range(16):
                for m in range(8):
                    rows = jnp.full((16,), 4 * ggl + m // 2, jnp.int32)
                    cols = iota + (16 * (m % 2))
                    ob[ggl, pl.ds(16 * m, 16)] = plsc.load_gather(tl_v, [rows, cols])
            pltpu.sync_copy(ob.at[pl.ds(0, 16)], out_hbm.at[pl.ds(G - 16, 16)])

    return k(table_t, tail64)


def _sc_gather_sum(table128, pieces):
    mesh = plsc.VectorSubcoreMesh(
        core_axis_name="c", subcore_axis_name="s", num_cores=NC, num_subcores=NS
    )

    @functools.partial(
        pl.kernel,
        out_type=jax.ShapeDtypeStruct((B, D), jnp.float32),
        mesh=mesh,
        scratch_types=[
            pltpu.VMEM((BW, S), jnp.int32),
            pltpu.VMEM((NBUF, 112), jnp.int32),
            pltpu.VMEM((HALF, 128), jnp.float32),
            pltpu.VMEM((HALF, 128), jnp.float32),
            pltpu.VMEM((HALF, 128), jnp.float32),
            pltpu.VMEM((HALF, 128), jnp.float32),
            pltpu.VMEM((BW, D), jnp.float32),
            pltpu.SemaphoreType.DMA,
            pltpu.SemaphoreType.DMA,
            pltpu.SemaphoreType.DMA,
            pltpu.SemaphoreType.DMA,
        ],
        compiler_params=pltpu.CompilerParams(use_tc_tiling_on_sc=True),
    )
    def k(table_hbm, idx_hbm, out_hbm, idx_v, gring, r0, r1, r2, r3, feat_v,
          sem0, sem1, sem2, sem3):
        rows = (r0, r1, r2, r3)
        sems = (sem0, sem1, sem2, sem3)
        wid = lax.axis_index("s") * NC + lax.axis_index("c")
        base = wid * BW
        pltpu.sync_copy(idx_hbm.at[pl.ds(base, BW)], idx_v)

        @pl.loop(0, BW)
        def _(r):
            zero = jnp.zeros((16,), jnp.float32)
            feat_v[r, pl.ds(0, 16)] = zero
            feat_v[r, pl.ds(16, 16)] = zero

        starts = (0, 16, 32, 48, 64, 80, 84)

        def g_compute(hr, buf):
            r = hr // 2
            cbase = (buf % 2) * HALF
            for st in starts:
                v = idx_v[r, pl.ds(cbase + st, 16)]
                gring[buf, pl.ds(st, 16)] = lax.shift_right_logical(v, 2)

        def issue(buf):
            pltpu.async_copy(
                table_hbm.at[gring.at[buf, pl.ds(0, HALF)]], rows[buf], sems[buf]
            )

        def wait(buf):
            pltpu.make_async_copy(
                table_hbm.at[gring.at[buf, pl.ds(0, HALF)]], rows[buf], sems[buf]
            ).wait()

        def accumulate(hr, buf):
            r = hr // 2
            cbase = (buf % 2) * HALF
            rv = rows[buf]
            zero = jnp.zeros((16,), jnp.float32)
            acc_a = [zero, zero, zero, zero]
            acc_b = [zero, zero, zero, zero]
            for w, st in enumerate(starts):
                iv = idx_v[r, pl.ds(cbase + st, 16)]
                offv = (iv & 3) * 32
                offv2 = offv + 16
                for j in range(12 if w == 6 else 0, 16):
                    jg = st + j
                    k4 = jg % 4
                    o1 = pl.multiple_of(offv[j], 16)
                    o2 = pl.multiple_of(offv2[j], 16)
                    acc_a[k4] = acc_a[k4] + rv[jg, pl.ds(o1, 16)]
                    acc_b[k4] = acc_b[k4] + rv[jg, pl.ds(o2, 16)]
            feat_v[r, pl.ds(0, 16)] += (acc_a[0] + acc_a[1]) + (acc_a[2] + acc_a[3])
            feat_v[r, pl.ds(16, 16)] += (acc_b[0] + acc_b[1]) + (acc_b[2] + acc_b[3])

        for buf in range(NBUF):
            g_compute(buf, buf)
            issue(buf)

        @pl.loop(0, NHR - NBUF, step=NBUF)
        def _(hr0):
            for buf in range(NBUF):
                hr = hr0 + buf
                wait(buf)
                accumulate(hr, buf)
                g_compute(hr + NBUF, buf)
                issue(buf)

        for buf in range(NBUF):
            wait(buf)
            accumulate(NHR - NBUF + buf, buf)

        pltpu.sync_copy(feat_v, out_hbm.at[pl.ds(base, BW)])

    return k(table128, pieces)


def _tc_tail(feature, pieces, pos, neg, w, b2, t0):
    BB = 256

    def body(feat_ref, pieces_ref, pos_ref, neg_ref, w_ref, b_ref, t0_ref, out_ref):
        cnt0 = jnp.sum((pieces_ref[...] == 0).astype(jnp.float32), axis=1)
        feat = feat_ref[...] - cnt0[:, None] * t0_ref[...]
        logits = lax.dot_general(
            feat, w_ref[...], (((1,), (1,)), ((), ())),
            preferred_element_type=jnp.float32,
        ) + b_ref[...]
        trans = 1.0 / (1.0 + jnp.exp(-logits))
        p = pos_ref[...]
        n = neg_ref[...]
        out_ref[...] = jnp.sum((p - n) * trans + n, axis=1)

    return pl.pallas_call(
        body,
        grid=(B // BB,),
        in_specs=[
            pl.BlockSpec((BB, D), lambda i: (i, 0)),
            pl.BlockSpec((BB, S), lambda i: (i, 0)),
            pl.BlockSpec((BB, T), lambda i: (i, 0)),
            pl.BlockSpec((BB, T), lambda i: (i, 0)),
            pl.BlockSpec((T, D), lambda i: (0, 0)),
            pl.BlockSpec((1, T), lambda i: (0, 0)),
            pl.BlockSpec((1, D), lambda i: (0, 0)),
        ],
        out_specs=pl.BlockSpec((BB,), lambda i: (i,)),
        out_shape=jax.ShapeDtypeStruct((B,), jnp.float32),
    )(feature, pieces, pos, neg, w, b2, t0)


def kernel(pieces, tree_pos_path, tree_neg_path, emb_table, W, b):
    pieces = pieces.astype(jnp.int32)
    table_t = emb_table.T
    tail64 = lax.slice(emb_table, (VOCAB - 64, 0), (VOCAB, D))
    table128 = _sc_regroup(table_t, tail64)
    feature = _sc_gather_sum(table128, pieces)
    b2 = b.reshape(1, T)
    t0 = lax.slice(emb_table, (0, 0), (1, D))
    return _tc_tail(feature, pieces, tree_pos_path, tree_neg_path, W, b2, t0)


# speedup vs baseline: 1.6412x; 1.6412x over previous
import functools

import jax
import jax.numpy as jnp
from jax import lax
from jax.experimental import pallas as pl
from jax.experimental.pallas import tpu as pltpu
from jax.experimental.pallas import tpu_sc as plsc

VOCAB = 1000000
D = 32
B = 4096
S = 200
T = 1024
NC, NS = 2, 16
NW = NC * NS
BW = B // NW
HALF = S // 2
NHR = 2 * BW
NBUF = 4
G = VOCAB // 4
NBLK = VOCAB // 128
BLK_PER_W = NBLK // NW
BLK_REM = NBLK % NW


def _sc_regroup(table_t, tail64):
    mesh = plsc.VectorSubcoreMesh(
        core_axis_name="c", subcore_axis_name="s", num_cores=NC, num_subcores=NS
    )

    @functools.partial(
        pl.kernel,
        out_type=jax.ShapeDtypeStruct((G, 128), jnp.float32),
        mesh=mesh,
        scratch_types=[
            pltpu.VMEM((32, 128), jnp.float32),
            pltpu.VMEM((32, 128), jnp.float32),
            pltpu.VMEM((32, 128), jnp.float32),
            pltpu.VMEM((32, 128), jnp.float32),
            pltpu.VMEM((64, 32), jnp.float32),
            pltpu.SemaphoreType.DMA,
            pltpu.SemaphoreType.DMA,
            pltpu.SemaphoreType.DMA,
            pltpu.SemaphoreType.DMA,
        ],
        compiler_params=pltpu.CompilerParams(
            use_tc_tiling_on_sc=True, needs_layout_passes=False
        ),
    )
    def k(tt_hbm, tail_hbm, out_hbm, w0, w1, o0, o1, tl_v,
          gsem0, gsem1, osem0, osem1):
        wins = (w0, w1)
        obufs = (o0, o1)
        gsems = (gsem0, gsem1)
        osems = (osem0, osem1)
        iota = lax.iota(jnp.int32, 16)
        wid = lax.axis_index("s") * NC + lax.axis_index("c")
        blk0 = wid * BLK_PER_W

        def issue_in(blk, buf):
            pltpu.async_copy(
                tt_hbm.at[:, pl.ds(blk * 128, 128)], wins[buf], gsems[buf]
            )

        def wait_in(buf):
            pltpu.make_async_copy(
                tt_hbm.at[:, pl.ds(0, 128)], wins[buf], gsems[buf]
            ).wait()

        def transpose_block(buf):
            w = wins[buf]
            ob = obufs[buf]
            zeros16 = jnp.zeros((16,), jnp.int32)

            @plsc.parallel_loop(0, 32, 1, unroll=4)
            def _(ggl):
                for m in range(8):
                    rows = iota + (16 * (m % 2))
                    cols = (4 * ggl + m // 2) + zeros16
                    ob[ggl, pl.ds(16 * m, 16)] = plsc.load_gather(w, [rows, cols])

        def issue_out(blk, buf):
            pltpu.async_copy(
                obufs[buf], out_hbm.at[pl.ds(blk * 32, 32)], osems[buf]
            )

        def wait_out(buf):
            pltpu.make_async_copy(
                obufs[buf], out_hbm.at[pl.ds(0, 32)], osems[buf]
            ).wait()

        issue_in(blk0, 0)
        issue_in(blk0 + 1, 1)
        for i in range(2):
            wait_in(i)
            transpose_block(i)
            issue_out(blk0 + i, i)
            issue_in(blk0 + i + 2, i)

        @pl.loop(0, BLK_PER_W - 4, step=2)
        def _(j0):
            i = j0 + 2
            for buf in range(2):
                wait_in(buf)
                wait_out(buf)
                transpose_block(buf)
                issue_out(blk0 + i + buf, buf)
                issue_in(blk0 + i + buf + 2, buf)

        for buf in range(2):
            wait_in(buf)
            wait_out(buf)
            transpose_block(buf)
            issue_out(blk0 + BLK_PER_W - 2 + buf, buf)
        wait_out(0)
        wait_out(1)

        @pl.when(wid < NBLK - NW * BLK_PER_W)
        def _():
            blk = NW * BLK_PER_W + wid
            issue_in(blk, 0)
            wait_in(0)
            transpose_block(0)
            issue_out(blk, 0)
            wait_out(0)

        @pl.when(wid == 0)
        def _():
            pltpu.sync_copy(tail_hbm, tl_v)
            ob = obufs[0]
            for ggl in range(16):
                for m in range(8):
                    rows = jnp.full((16,), 4 * ggl + m // 2, jnp.int32)
                    cols = iota + (16 * (m % 2))
                    ob[ggl, pl.ds(16 * m, 16)] = plsc.load_gather(tl_v, [rows, cols])
            pltpu.sync_copy(ob.at[pl.ds(0, 16)], out_hbm.at[pl.ds(G - 16, 16)])

    return k(table_t, tail64)


def _sc_gather_sum(table128, pieces):
    mesh = plsc.VectorSubcoreMesh(
        core_axis_name="c", subcore_axis_name="s", num_cores=NC, num_subcores=NS
    )

    @functools.partial(
        pl.kernel,
        out_type=jax.ShapeDtypeStruct((B, D), jnp.float32),
        mesh=mesh,
        scratch_types=[
            pltpu.VMEM((BW, S), jnp.int32),
            pltpu.VMEM((NBUF, 112), jnp.int32),
            pltpu.VMEM((HALF, 128), jnp.float32),
            pltpu.VMEM((HALF, 128), jnp.float32),
            pltpu.VMEM((HALF, 128), jnp.float32),
            pltpu.VMEM((HALF, 128), jnp.float32),
            pltpu.VMEM((BW, D), jnp.float32),
            pltpu.SemaphoreType.DMA,
            pltpu.SemaphoreType.DMA,
            pltpu.SemaphoreType.DMA,
            pltpu.SemaphoreType.DMA,
        ],
        compiler_params=pltpu.CompilerParams(use_tc_tiling_on_sc=True),
    )
    def k(table_hbm, idx_hbm, out_hbm, idx_v, gring, r0, r1, r2, r3, feat_v,
          sem0, sem1, sem2, sem3):
        rows = (r0, r1, r2, r3)
        sems = (sem0, sem1, sem2, sem3)
        wid = lax.axis_index("s") * NC + lax.axis_index("c")
        base = wid * BW
        pltpu.sync_copy(idx_hbm.at[pl.ds(base, BW)], idx_v)

        @pl.loop(0, BW)
        def _(r):
            zero = jnp.zeros((16,), jnp.float32)
            feat_v[r, pl.ds(0, 16)] = zero
            feat_v[r, pl.ds(16, 16)] = zero

        starts = (0, 16, 32, 48, 64, 80, 84)

        def g_compute(hr, buf):
            r = hr // 2
            cbase = (buf % 2) * HALF
            for st in starts:
                v = idx_v[r, pl.ds(cbase + st, 16)]
                gring[buf, pl.ds(st, 16)] = lax.shift_right_logical(v, 2)

        def issue(buf):
            pltpu.async_copy(
                table_hbm.at[gring.at[buf, pl.ds(0, HALF)]], rows[buf], sems[buf]
            )

        def wait(buf):
            pltpu.make_async_copy(
                table_hbm.at[gring.at[buf, pl.ds(0, HALF)]], rows[buf], sems[buf]
            ).wait()

        def accumulate(hr, buf):
            r = hr // 2
            cbase = (buf % 2) * HALF
            rv = rows[buf]
            zero = jnp.zeros((16,), jnp.float32)
            acc_a = [zero, zero, zero, zero]
            acc_b = [zero, zero, zero, zero]
            for w, st in enumerate(starts):
                iv = idx_v[r, pl.ds(cbase + st, 16)]
                offv = (iv & 3) * 32
                offv2 = offv + 16
                for j in range(12 if w == 6 else 0, 16):
                    jg = st + j
                    k4 = jg % 4
                    o1 = pl.multiple_of(offv[j], 16)
                    o2 = pl.multiple_of(offv2[j], 16)
                    acc_a[k4] = acc_a[k4] + rv[jg, pl.ds(o1, 16)]
                    acc_b[k4] = acc_b[k4] + rv[jg, pl.ds(o2, 16)]
            feat_v[r, pl.ds(0, 16)] += (acc_a[0] + acc_a[1]) + (acc_a[2] + acc_a[3])
            feat_v[r, pl.ds(16, 16)] += (acc_b[0] + acc_b[1]) + (acc_b[2] + acc_b[3])

        for buf in range(NBUF):
            g_compute(buf, buf)
            issue(buf)

        @pl.loop(0, NHR - NBUF, step=NBUF)
        def _(hr0):
            for buf in range(NBUF):
                hr = hr0 + buf
                wait(buf)
                accumulate(hr, buf)
                g_compute(hr + NBUF, buf)
                issue(buf)

        for buf in range(NBUF):
            wait(buf)
            accumulate(NHR - NBUF + buf, buf)

        pltpu.sync_copy(feat_v, out_hbm.at[pl.ds(base, BW)])

    return k(table128, pieces)


def _tc_tail(feature, pieces, pos, neg, w, b2, t0):
    BB = 256

    def body(feat_ref, pieces_ref, pos_ref, neg_ref, w_ref, b_ref, t0_ref, out_ref):
        cnt0 = jnp.sum((pieces_ref[...] == 0).astype(jnp.float32), axis=1)
        feat = feat_ref[...] - cnt0[:, None] * t0_ref[...]
        logits = lax.dot_general(
            feat, w_ref[...], (((1,), (1,)), ((), ())),
            preferred_element_type=jnp.float32,
        ) + b_ref[...]
        trans = 1.0 / (1.0 + jnp.exp(-logits))
        p = pos_ref[...]
        n = neg_ref[...]
        out_ref[...] = jnp.sum((p - n) * trans + n, axis=1)

    return pl.pallas_call(
        body,
        grid=(B // BB,),
        in_specs=[
            pl.BlockSpec((BB, D), lambda i: (i, 0)),
            pl.BlockSpec((BB, S), lambda i: (i, 0)),
            pl.BlockSpec((BB, T), lambda i: (i, 0)),
            pl.BlockSpec((BB, T), lambda i: (i, 0)),
            pl.BlockSpec((T, D), lambda i: (0, 0)),
            pl.BlockSpec((1, T), lambda i: (0, 0)),
            pl.BlockSpec((1, D), lambda i: (0, 0)),
        ],
        out_specs=pl.BlockSpec((BB,), lambda i: (i,)),
        out_shape=jax.ShapeDtypeStruct((B,), jnp.float32),
    )(feature, pieces, pos, neg, w, b2, t0)


def kernel(pieces, tree_pos_path, tree_neg_path, emb_table, W, b):
    pieces = pieces.astype(jnp.int32)
    table_t = emb_table.T
    tail64 = lax.slice(emb_table, (VOCAB - 64, 0), (VOCAB, D))
    table128 = _sc_regroup(table_t, tail64)
    feature = _sc_gather_sum(table128, pieces)
    b2 = b.reshape(1, T)
    t0 = lax.slice(emb_table, (0, 0), (1, D))
    return _tc_tail(feature, pieces, tree_pos_path, tree_neg_path, W, b2, t0)
